# baseline (speedup 1.0000x reference)
import functools

import jax
import jax.numpy as jnp
from jax.experimental import pallas as pl
from jax.experimental.pallas import tpu as pltpu

_LANES = 128
_N_SUB = 2
_K_SPLIT = 2


def _bnneck_kernel(x_ref, w_ref, gamma_ref, beta_ref, o_ref, x_asm, acc,
                   *, n, c_in):
    kc = c_in // _K_SPLIT
    sub = w_ref.shape[0]
    k = pl.program_id(1)
    s = pl.program_id(2)

    @pl.when(s == 0)
    def _cache_x():
        x_asm[:, pl.ds(k * kc, kc)] = (
            x_ref[...].astype(jnp.bfloat16).reshape(n, kc))

    wk = w_ref[...].astype(jnp.bfloat16).reshape(sub, kc)
    y = jax.lax.dot_general(
        x_asm[:, pl.ds(k * kc, kc)], wk,
        dimension_numbers=(((1,), (1,)), ((), ())),
        preferred_element_type=jnp.float32)

    @pl.when(k == 0)
    def _first():
        acc[s] = y

    @pl.when(k == _K_SPLIT - 1)
    def _last():
        yf = y if _K_SPLIT == 1 else acc[s] + y
        inv_n = 1.0 / float(n)
        mean = jnp.sum(yf, axis=0, keepdims=True) * inv_n
        diff = yf - mean
        var = jnp.sum(diff * diff, axis=0, keepdims=True) * inv_n
        z = diff * jax.lax.rsqrt(var + 1e-5)
        z = z * gamma_ref[...] + beta_ref[...]
        o_ref[:, pl.ds(s * sub, sub)] = jnp.where(z >= 0, z, 0.25 * z)


def kernel(x, weight, gamma, beta):
    n, c_in, h, w_sp = x.shape
    assert h == 1 and w_sp == 1
    c_out = weight.shape[0]
    assert n % 8 == 0 and c_in % (_K_SPLIT * _LANES) == 0
    kj = c_in // _LANES
    kjs = kj // _K_SPLIT
    tile_co = c_out // (2 * _N_SUB)
    assert tile_co % _LANES == 0

    x3 = x.reshape(n, kj, _LANES)
    w3 = weight.reshape(c_out, kj, _LANES)
    gamma2 = gamma.reshape(1, c_out).astype(jnp.float32)
    beta2 = beta.reshape(1, c_out).astype(jnp.float32)

    body = functools.partial(_bnneck_kernel, n=n, c_in=c_in)
    return pl.pallas_call(
        body,
        out_shape=jax.ShapeDtypeStruct((n, c_out), x.dtype),
        grid=(2, _K_SPLIT, _N_SUB),
        in_specs=[
            pl.BlockSpec((n, kjs, _LANES), lambda i, k, s: (0, k, 0)),
            pl.BlockSpec((tile_co, kjs, _LANES),
                         lambda i, k, s: (i * _N_SUB + s, k, 0)),
            pl.BlockSpec((1, tile_co), lambda i, k, s: (0, i * _N_SUB + s)),
            pl.BlockSpec((1, tile_co), lambda i, k, s: (0, i * _N_SUB + s)),
        ],
        out_specs=pl.BlockSpec((n, _N_SUB * tile_co), lambda i, k, s: (0, i)),
        scratch_shapes=[
            pltpu.VMEM((n, c_in), jnp.bfloat16),
            pltpu.VMEM((_N_SUB, n, tile_co), jnp.float32),
        ],
        compiler_params=pltpu.CompilerParams(
            dimension_semantics=("parallel", "arbitrary", "arbitrary"),
            vmem_limit_bytes=56 * 1024 * 1024,
        ),
    )(x3, w3, gamma2, beta2)


# speedup vs baseline: 1.1570x; 1.1570x over previous
import functools

import jax
import jax.numpy as jnp
from jax.experimental import pallas as pl
from jax.experimental.pallas import tpu as pltpu

_LANES = 128
_N_SUB = 2


def _bnneck_kernel(x_ref, w_ref, gamma_ref, beta_ref, o_ref, x_asm, *, n):
    c_in = x_ref.shape[1] * _LANES

    @pl.when(pl.program_id(1) == 0)
    def _cache_x():
        x_asm[...] = x_ref[...].astype(jnp.bfloat16).reshape(n, c_in)

    wk = w_ref[...].astype(jnp.bfloat16).reshape(w_ref.shape[0], c_in)
    y = jax.lax.dot_general(
        x_asm[...], wk, dimension_numbers=(((1,), (1,)), ((), ())),
        preferred_element_type=jnp.float32)
    inv_n = 1.0 / float(n)
    mean = jnp.sum(y, axis=0, keepdims=True) * inv_n
    diff = y - mean
    var = jnp.sum(diff * diff, axis=0, keepdims=True) * inv_n
    z = diff * jax.lax.rsqrt(var + 1e-5)
    z = z * gamma_ref[...] + beta_ref[...]
    o_ref[...] = jnp.where(z >= 0, z, 0.25 * z)


def kernel(x, weight, gamma, beta):
    n, c_in, h, w_sp = x.shape
    assert h == 1 and w_sp == 1
    c_out = weight.shape[0]
    assert n % 8 == 0 and c_in % _LANES == 0
    kj = c_in // _LANES
    tile_co = c_out // (2 * _N_SUB)
    assert tile_co % _LANES == 0

    x3 = x.reshape(n, kj, _LANES)
    w3 = weight.reshape(c_out, kj, _LANES)
    gamma2 = gamma.reshape(1, c_out).astype(jnp.float32)
    beta2 = beta.reshape(1, c_out).astype(jnp.float32)

    body = functools.partial(_bnneck_kernel, n=n)
    return pl.pallas_call(
        body,
        out_shape=jax.ShapeDtypeStruct((n, c_out), x.dtype),
        grid=(2, _N_SUB),
        in_specs=[
            pl.BlockSpec((n, kj, _LANES), lambda i, j: (0, 0, 0)),
            pl.BlockSpec((tile_co, kj, _LANES),
                         lambda i, j: (i * _N_SUB + j, 0, 0)),
            pl.BlockSpec((1, tile_co), lambda i, j: (0, i * _N_SUB + j)),
            pl.BlockSpec((1, tile_co), lambda i, j: (0, i * _N_SUB + j)),
        ],
        out_specs=pl.BlockSpec((n, tile_co), lambda i, j: (0, i * _N_SUB + j)),
        scratch_shapes=[pltpu.VMEM((n, c_in), jnp.bfloat16)],
        compiler_params=pltpu.CompilerParams(
            dimension_semantics=("parallel", "arbitrary"),
            vmem_limit_bytes=56 * 1024 * 1024,
        ),
    )(x3, w3, gamma2, beta2)
